# baseline (speedup 1.0000x reference)
import functools

import jax
import jax.numpy as jnp
from jax import lax
from jax.experimental import pallas as pl
from jax.experimental.pallas import tpu as pltpu
from jax.experimental.pallas import tpu_sc as plsc

BATCH = 16384
D = 128
NC = 2
NS = 16
NW = NC * NS
BPW = BATCH // NW
CH = 128
NCH = BPW // CH
NCHUNKS = 2 * NCH
NBUF = 7

_mesh = plsc.VectorSubcoreMesh(core_axis_name="c", subcore_axis_name="s")


@functools.partial(
    pl.kernel,
    mesh=_mesh,
    out_type=(
        jax.ShapeDtypeStruct((BATCH, D), jnp.float32),
        jax.ShapeDtypeStruct((BATCH, D), jnp.float32),
    ),
    scratch_types=[
        pltpu.VMEM((NCH, CH), jnp.int32),
        pltpu.VMEM((NCH, CH), jnp.int32),
        pltpu.VMEM((NBUF, CH, D), jnp.float32),
        pltpu.SemaphoreType.DMA((2,)),
        pltpu.SemaphoreType.DMA((NCHUNKS,)),
        pltpu.SemaphoreType.DMA((NCHUNKS,)),
    ],
)
def _sc_gather(uids, iids, utab, itab, uout, iout, uidx_v, iidx_v,
               bufs, isem, gsem, wsem):
    wid = lax.axis_index("s") * NC + lax.axis_index("c")
    base = wid * BPW

    ldu = pltpu.async_copy(uids.at[wid], uidx_v, isem.at[0])
    ldi = pltpu.async_copy(iids.at[wid], iidx_v, isem.at[1])
    ldu.wait()
    ldi.wait()

    def chunk(c):
        j = c // 2
        if c % 2 == 0:
            return utab, uidx_v.at[j], uout, base + j * CH
        return itab, iidx_v.at[j], iout, base + j * CH

    def fire(c):
        tab, idx, _, _ = chunk(c)
        return pltpu.async_copy(tab.at[idx], bufs.at[c % NBUF], gsem.at[c])

    DEPTH = 2
    gathers = [fire(c) for c in range(DEPTH)]
    writebacks = []
    for c in range(NCHUNKS):
        _, _, out, off = chunk(c)
        gathers[c].wait()
        writebacks.append(pltpu.async_copy(bufs.at[c % NBUF],
                                           out.at[pl.ds(off, CH)], wsem.at[c]))
        nxt = c + DEPTH
        if nxt < NCHUNKS:
            if nxt >= NBUF:
                writebacks[nxt - NBUF].wait()
            gathers.append(fire(nxt))

    for c in range(NCHUNKS - NBUF, NCHUNKS):
        writebacks[c].wait()


def kernel(user_ids, item_ids, user_table, item_table):
    uids = user_ids.astype(jnp.int32).reshape(NW, NCH, CH)
    iids = item_ids.astype(jnp.int32).reshape(NW, NCH, CH)
    return _sc_gather(uids, iids, user_table, item_table)


# speedup vs baseline: 1.0207x; 1.0207x over previous
import functools

import jax
import jax.numpy as jnp
from jax import lax
from jax.experimental import pallas as pl
from jax.experimental.pallas import tpu as pltpu
from jax.experimental.pallas import tpu_sc as plsc

BATCH = 16384
D = 128
NC = 2
NS = 16
NW = NC * NS
BPW = BATCH // NW
CH = 128
NCH = BPW // CH
NCHUNKS = 2 * NCH
NBUF = 7

_mesh = plsc.VectorSubcoreMesh(core_axis_name="c", subcore_axis_name="s")


@functools.partial(
    pl.kernel,
    mesh=_mesh,
    out_type=(
        jax.ShapeDtypeStruct((BATCH, D), jnp.float32),
        jax.ShapeDtypeStruct((BATCH, D), jnp.float32),
    ),
    scratch_types=[
        pltpu.VMEM((NCH, CH), jnp.int32),
        pltpu.VMEM((NCH, CH), jnp.int32),
        pltpu.VMEM((NBUF, CH, D), jnp.float32),
        pltpu.SemaphoreType.DMA((2,)),
        pltpu.SemaphoreType.DMA((NCHUNKS,)),
        pltpu.SemaphoreType.DMA((NCHUNKS,)),
    ],
)
def _sc_gather(uids, iids, utab, itab, uout, iout, uidx_v, iidx_v,
               bufs, isem, gsem, wsem):
    wid = lax.axis_index("s") * NC + lax.axis_index("c")
    base = wid * BPW

    ldu = pltpu.async_copy(uids.at[wid], uidx_v, isem.at[0])
    ldi = pltpu.async_copy(iids.at[wid], iidx_v, isem.at[1])
    ldu.wait()
    ldi.wait()

    def chunk(c):
        j = c // 2
        if c % 2 == 0:
            return utab, uidx_v.at[j], uout, base + j * CH
        return itab, iidx_v.at[j], iout, base + j * CH

    def fire(c):
        tab, idx, _, _ = chunk(c)
        return pltpu.async_copy(tab.at[idx], bufs.at[c % NBUF], gsem.at[c])

    DEPTH = 4
    gathers = [fire(c) for c in range(DEPTH)]
    writebacks = []
    for c in range(NCHUNKS):
        _, _, out, off = chunk(c)
        gathers[c].wait()
        writebacks.append(pltpu.async_copy(bufs.at[c % NBUF],
                                           out.at[pl.ds(off, CH)], wsem.at[c]))
        nxt = c + DEPTH
        if nxt < NCHUNKS:
            if nxt >= NBUF:
                writebacks[nxt - NBUF].wait()
            gathers.append(fire(nxt))

    for c in range(NCHUNKS - NBUF, NCHUNKS):
        writebacks[c].wait()


def kernel(user_ids, item_ids, user_table, item_table):
    uids = user_ids.astype(jnp.int32).reshape(NW, NCH, CH)
    iids = item_ids.astype(jnp.int32).reshape(NW, NCH, CH)
    return _sc_gather(uids, iids, user_table, item_table)


# speedup vs baseline: 1.0456x; 1.0243x over previous
import functools

import jax
import jax.numpy as jnp
from jax import lax
from jax.experimental import pallas as pl
from jax.experimental.pallas import tpu as pltpu
from jax.experimental.pallas import tpu_sc as plsc

BATCH = 16384
D = 128
NC = 2
NS = 16
NW = NC * NS
BPW = BATCH // NW
CH = 128
NCH = BPW // CH
NCHUNKS = 2 * NCH
NBUF = 7

_mesh = plsc.VectorSubcoreMesh(core_axis_name="c", subcore_axis_name="s")


@functools.partial(
    pl.kernel,
    mesh=_mesh,
    out_type=(
        jax.ShapeDtypeStruct((BATCH, D), jnp.float32),
        jax.ShapeDtypeStruct((BATCH, D), jnp.float32),
    ),
    scratch_types=[
        pltpu.VMEM((NCH, CH), jnp.int32),
        pltpu.VMEM((NCH, CH), jnp.int32),
        pltpu.VMEM((NBUF, CH, D), jnp.float32),
        pltpu.SemaphoreType.DMA((2,)),
        pltpu.SemaphoreType.DMA((NCHUNKS,)),
        pltpu.SemaphoreType.DMA((NCHUNKS,)),
    ],
)
def _sc_gather(uids, iids, utab, itab, uout, iout, uidx_v, iidx_v,
               bufs, isem, gsem, wsem):
    wid = lax.axis_index("s") * NC + lax.axis_index("c")
    base = wid * BPW

    ldu = pltpu.async_copy(uids.at[wid], uidx_v, isem.at[0])
    ldi = pltpu.async_copy(iids.at[wid], iidx_v, isem.at[1])
    ldu.wait()
    ldi.wait()

    def chunk(c):
        j = c // 2
        if c % 2 == 0:
            return utab, uidx_v.at[j], uout, base + j * CH
        return itab, iidx_v.at[j], iout, base + j * CH

    def fire(c):
        tab, idx, _, _ = chunk(c)
        return pltpu.async_copy(tab.at[idx], bufs.at[c % NBUF], gsem.at[c])

    DEPTH = NBUF
    gathers = [fire(c) for c in range(DEPTH)]
    writebacks = []
    for c in range(NCHUNKS):
        _, _, out, off = chunk(c)
        gathers[c].wait()
        writebacks.append(pltpu.async_copy(bufs.at[c % NBUF],
                                           out.at[pl.ds(off, CH)], wsem.at[c]))
        nxt = c + DEPTH
        if nxt < NCHUNKS:
            if nxt >= NBUF:
                writebacks[nxt - NBUF].wait()
            gathers.append(fire(nxt))

    for c in range(NCHUNKS - NBUF, NCHUNKS):
        writebacks[c].wait()


def kernel(user_ids, item_ids, user_table, item_table):
    uids = user_ids.astype(jnp.int32).reshape(NW, NCH, CH)
    iids = item_ids.astype(jnp.int32).reshape(NW, NCH, CH)
    return _sc_gather(uids, iids, user_table, item_table)
